# baseline (speedup 1.0000x reference)
import jax
import jax.numpy as jnp
from jax import lax
from jax.experimental import pallas as pl
from jax.experimental.pallas import tpu as pltpu
from jax.experimental.pallas import tpu_sc as plsc

B = 16384
CTX = 20
D = 64
V = 1000000
NC = 2
NS = 16
NW = NC * NS
BPW = B // NW
CB = 64
NCHUNK = BPW // CB

_PERM_DN = lax.GatherDimensionNumbers(
    offset_dims=(), collapsed_slice_dims=(0,), start_index_map=(0,))


def _shuf(v, idx):
    return lax.gather(v, idx[:, None], _PERM_DN, (1,),
                      mode=lax.GatherScatterMode.PROMISE_IN_BOUNDS)


def _hsum(v, perms):
    for p in perms:
        v = v + _shuf(v, p)
    return v


def _hmax(v, perms):
    for p in perms:
        v = jnp.maximum(v, _shuf(v, p))
    return v


def _dot64(t_rows, c_rows, b, w, perms):
    acc = None
    for h in range(2):
        tu = plsc.unpack(t_rows[b, pl.ds(32 * h, 32)],
                         format=plsc.PackFormat.INTERLEAVED)
        cu = plsc.unpack(c_rows[w, b, pl.ds(32 * h, 32)],
                         format=plsc.PackFormat.INTERLEAVED)
        part = tu[0] * cu[0] + tu[1] * cu[1]
        acc = part if acc is None else acc + part
    return _hsum(acc, perms)


def _sc_body(tgt_hbm, ctxT_hbm, table_hbm, a_hbm, z_hbm,
             tgt_idx, ctx_idx, tgt_rows, ctx_rows, a_stage, z_stage,
             sem_t, sem_c):
    wid = lax.axis_index("s") * NC + lax.axis_index("c")
    lane = lax.iota(jnp.int32, 16)
    lane0 = lane == 0
    perms = [lane ^ k for k in (8, 4, 2, 1)]
    neg_inf = jnp.float32(-jnp.inf)

    @pl.loop(0, NCHUNK)
    def _chunk(c):
        base = wid * BPW + c * CB

        pltpu.sync_copy(tgt_hbm.at[pl.ds(base, CB)], tgt_idx)
        for w in range(CTX):
            pltpu.sync_copy(ctxT_hbm.at[w, pl.ds(base, CB)], ctx_idx.at[w])

        tcopy = pltpu.async_copy(table_hbm.at[tgt_idx], tgt_rows, sem_t)
        ccopies = [
            pltpu.async_copy(table_hbm.at[ctx_idx.at[w]], ctx_rows.at[w],
                             sem_c)
            for w in range(CTX)
        ]
        tcopy.wait()
        for cc in ccopies:
            cc.wait()

        @pl.loop(0, CB)
        def _row(b):
            s1_vec = None
            sv0 = jnp.full((16,), neg_inf, jnp.float32)
            sv1 = jnp.full((16,), neg_inf, jnp.float32)
            for w in range(CTX):
                s = _dot64(tgt_rows, ctx_rows, b, w, perms)
                if w == 1:
                    s1_vec = s
                if w < 16:
                    sv0 = jnp.where(lane == w, s, sv0)
                else:
                    sv1 = jnp.where(lane == (w - 16), s, sv1)
            m = _hmax(jnp.maximum(sv0, sv1), perms)
            z = _hsum(jnp.exp(sv0 - m) + jnp.exp(sv1 - m), perms)
            idxv = jnp.full((16,), b, jnp.int32)
            plsc.store_scatter(a_stage, [idxv], m - s1_vec, mask=lane0)
            plsc.store_scatter(z_stage, [idxv], z, mask=lane0)

        pltpu.sync_copy(a_stage, a_hbm.at[pl.ds(base, CB)])
        pltpu.sync_copy(z_stage, z_hbm.at[pl.ds(base, CB)])


def _finish_body(a_ref, z_ref, o_ref):
    o_ref[0, 0] = jnp.sum(a_ref[...] + jnp.log(z_ref[...])) * (1.0 / B)


@jax.jit
def kernel(target, context, in_embed, out_embed):
    table = jnp.concatenate(
        [out_embed.astype(jnp.bfloat16), in_embed.astype(jnp.bfloat16)],
        axis=0)
    tgt_off = target.astype(jnp.int32) + V
    ctx_t = context.astype(jnp.int32).T

    mesh = plsc.VectorSubcoreMesh(core_axis_name="c", subcore_axis_name="s")
    a, z = pl.kernel(
        _sc_body,
        out_type=(
            jax.ShapeDtypeStruct((B,), jnp.float32),
            jax.ShapeDtypeStruct((B,), jnp.float32),
        ),
        mesh=mesh,
        compiler_params=pltpu.CompilerParams(
            needs_layout_passes=False, use_tc_tiling_on_sc=False),
        scratch_types=[
            pltpu.VMEM((CB,), jnp.int32),
            pltpu.VMEM((CTX, CB), jnp.int32),
            pltpu.VMEM((CB, D), jnp.bfloat16),
            pltpu.VMEM((CTX, CB, D), jnp.bfloat16),
            pltpu.VMEM((CB,), jnp.float32),
            pltpu.VMEM((CB,), jnp.float32),
            pltpu.SemaphoreType.DMA,
            pltpu.SemaphoreType.DMA,
        ],
    )(tgt_off, ctx_t, table)

    loss = pl.pallas_call(
        _finish_body,
        out_shape=jax.ShapeDtypeStruct((1, 1), jnp.float32),
        out_specs=pl.BlockSpec(memory_space=pltpu.SMEM),
    )(a.reshape(128, 128), z.reshape(128, 128))
    return loss[0, 0]


# speedup vs baseline: 1.3558x; 1.3558x over previous
import jax
import jax.numpy as jnp
from jax import lax
from jax.experimental import pallas as pl
from jax.experimental.pallas import tpu as pltpu
from jax.experimental.pallas import tpu_sc as plsc

B = 16384
CTX = 20
D = 64
V = 1000000
NC = 2
NS = 16
NW = NC * NS
BPW = B // NW
CB = 64
NCHUNK = BPW // CB

_PERM_DN = lax.GatherDimensionNumbers(
    offset_dims=(), collapsed_slice_dims=(0,), start_index_map=(0,))


def _shuf(v, idx):
    return lax.gather(v, idx[:, None], _PERM_DN, (1,),
                      mode=lax.GatherScatterMode.PROMISE_IN_BOUNDS)


def _hsum(v, perms):
    for p in perms:
        v = v + _shuf(v, p)
    return v


def _hmax(v, perms):
    for p in perms:
        v = jnp.maximum(v, _shuf(v, p))
    return v


def _dot64(t_rows, c_rows, b, w, perms):
    acc = None
    for h in range(2):
        tu = plsc.unpack(t_rows[b, pl.ds(32 * h, 32)],
                         format=plsc.PackFormat.INTERLEAVED)
        cu = plsc.unpack(c_rows[w, b, pl.ds(32 * h, 32)],
                         format=plsc.PackFormat.INTERLEAVED)
        part = tu[0] * cu[0] + tu[1] * cu[1]
        acc = part if acc is None else acc + part
    return _hsum(acc, perms)


def _sc_body(tgt_hbm, ctxT_hbm, in_tab_hbm, out_tab_hbm, a_hbm, z_hbm,
             tgt_idx, ctx_idx, tgt_rows, ctx_rows, a_stage, z_stage,
             sem_t, sem_c):
    wid = lax.axis_index("s") * NC + lax.axis_index("c")
    lane = lax.iota(jnp.int32, 16)
    lane0 = lane == 0
    perms = [lane ^ k for k in (8, 4, 2, 1)]
    neg_inf = jnp.float32(-jnp.inf)

    @pl.loop(0, NCHUNK)
    def _chunk(c):
        base = wid * BPW + c * CB

        pltpu.sync_copy(tgt_hbm.at[pl.ds(base, CB)], tgt_idx)
        for w in range(CTX):
            pltpu.sync_copy(ctxT_hbm.at[w, pl.ds(base, CB)], ctx_idx.at[w])

        tcopy = pltpu.async_copy(in_tab_hbm.at[tgt_idx], tgt_rows, sem_t)
        ccopies = [
            pltpu.async_copy(out_tab_hbm.at[ctx_idx.at[w]], ctx_rows.at[w],
                             sem_c)
            for w in range(CTX)
        ]
        tcopy.wait()
        for cc in ccopies:
            cc.wait()

        @pl.loop(0, CB)
        def _row(b):
            s1_vec = None
            sv0 = jnp.full((16,), neg_inf, jnp.float32)
            sv1 = jnp.full((16,), neg_inf, jnp.float32)
            for w in range(CTX):
                s = _dot64(tgt_rows, ctx_rows, b, w, perms)
                if w == 1:
                    s1_vec = s
                if w < 16:
                    sv0 = jnp.where(lane == w, s, sv0)
                else:
                    sv1 = jnp.where(lane == (w - 16), s, sv1)
            m = _hmax(jnp.maximum(sv0, sv1), perms)
            z = _hsum(jnp.exp(sv0 - m) + jnp.exp(sv1 - m), perms)
            idxv = jnp.full((16,), b, jnp.int32)
            plsc.store_scatter(a_stage, [idxv], m - s1_vec, mask=lane0)
            plsc.store_scatter(z_stage, [idxv], z, mask=lane0)

        pltpu.sync_copy(a_stage, a_hbm.at[pl.ds(base, CB)])
        pltpu.sync_copy(z_stage, z_hbm.at[pl.ds(base, CB)])


def _finish_body(a_ref, z_ref, o_ref):
    o_ref[0, 0] = jnp.sum(a_ref[...] + jnp.log(z_ref[...])) * (1.0 / B)


@jax.jit
def kernel(target, context, in_embed, out_embed):
    in_tab = in_embed.astype(jnp.bfloat16)
    out_tab = out_embed.astype(jnp.bfloat16)
    tgt_off = target.astype(jnp.int32)
    ctx_t = context.astype(jnp.int32).T

    mesh = plsc.VectorSubcoreMesh(core_axis_name="c", subcore_axis_name="s")
    a, z = pl.kernel(
        _sc_body,
        out_type=(
            jax.ShapeDtypeStruct((B,), jnp.float32),
            jax.ShapeDtypeStruct((B,), jnp.float32),
        ),
        mesh=mesh,
        compiler_params=pltpu.CompilerParams(
            needs_layout_passes=False, use_tc_tiling_on_sc=False),
        scratch_types=[
            pltpu.VMEM((CB,), jnp.int32),
            pltpu.VMEM((CTX, CB), jnp.int32),
            pltpu.VMEM((CB, D), jnp.bfloat16),
            pltpu.VMEM((CTX, CB, D), jnp.bfloat16),
            pltpu.VMEM((CB,), jnp.float32),
            pltpu.VMEM((CB,), jnp.float32),
            pltpu.SemaphoreType.DMA,
            pltpu.SemaphoreType.DMA,
        ],
    )(tgt_off, ctx_t, in_tab, out_tab)

    loss = pl.pallas_call(
        _finish_body,
        out_shape=jax.ShapeDtypeStruct((1, 1), jnp.float32),
        out_specs=pl.BlockSpec(memory_space=pltpu.SMEM),
    )(a.reshape(128, 128), z.reshape(128, 128))
    return loss[0, 0]
